# baseline (speedup 1.0000x reference)
import functools

import jax
import jax.numpy as jnp
from jax import lax
from jax.experimental import pallas as pl
from jax.experimental.pallas import tpu as pltpu
from jax.experimental.pallas import tpu_sc as plsc

B = 16384
NV = 26
CARD = 100000
ED = 16
IN_DIM = NV * ED
H1 = 128
H2 = 64

NC = 2
NS = 16
NW = NC * NS
BW = B // NW
BH = B // 2
BWH = BH // NW
GROUP = 256
CPF = BWH // GROUP
NG = NV * CPF


def _sc_gather_body(table_hbm, xt_hbm, out_hbm, idx_v, rows_v, sem0, sem1):
    wid = lax.axis_index("s") * NC + lax.axis_index("c")
    b0 = wid * BWH

    pltpu.sync_copy(xt_hbm.at[:, pl.ds(wid * CPF, CPF)], idx_v)

    def fire(v, c, buf_ref, sem):
        for e in range(ED):
            pltpu.make_async_copy(
                table_hbm.at[v, e].at[idx_v.at[v, c]], buf_ref.at[e], sem
            ).start()

    def drain(v, c, buf_ref, sem):
        for e in range(ED):
            pltpu.make_async_copy(
                table_hbm.at[v, e].at[idx_v.at[v, c]], buf_ref.at[e], sem
            ).wait()

    def write(v, c, buf_ref):
        pltpu.sync_copy(
            buf_ref, out_hbm.at[v, :, pl.ds(b0 + c * GROUP, GROUP)]
        )

    fire(0, 0, rows_v.at[0], sem0)

    def pair(p, _):
        g0 = p * 2
        v0, c0 = lax.div(g0, CPF), lax.rem(g0, CPF)
        v1, c1 = lax.div(g0 + 1, CPF), lax.rem(g0 + 1, CPF)
        drain(v0, c0, rows_v.at[0], sem0)
        fire(v1, c1, rows_v.at[1], sem1)
        write(v0, c0, rows_v.at[0])
        drain(v1, c1, rows_v.at[1], sem1)

        @pl.when(p < NG // 2 - 1)
        def _():
            v2, c2 = lax.div(g0 + 2, CPF), lax.rem(g0 + 2, CPF)
            fire(v2, c2, rows_v.at[0], sem0)

        write(v1, c1, rows_v.at[1])
        return 0

    lax.fori_loop(0, NG // 2, pair, 0)


def _sc_gather(tableT, xt3):
    mesh = plsc.VectorSubcoreMesh(
        core_axis_name="c", subcore_axis_name="s", num_cores=NC, num_subcores=NS
    )
    fn = pl.kernel(
        _sc_gather_body,
        out_type=jax.ShapeDtypeStruct((NV, ED, BH), jnp.float32),
        mesh=mesh,
        compiler_params=pltpu.CompilerParams(use_tc_tiling_on_sc=False),
        scratch_types=[
            pltpu.VMEM((NV, CPF, GROUP), jnp.int32),
            pltpu.VMEM((2, ED, GROUP), jnp.float32),
            pltpu.SemaphoreType.DMA,
            pltpu.SemaphoreType.DMA,
        ],
    )
    return fn(tableT, xt3)


def _mlp_body(enc_ref, w1_ref, b1_ref, w2_ref, b2_ref, w3_ref, b3_ref, out_ref):
    h = jnp.dot(w1_ref[...], enc_ref[...], preferred_element_type=jnp.float32)
    h = jnp.maximum(h + b1_ref[...], 0.0)
    h = jnp.dot(w2_ref[...], h, preferred_element_type=jnp.float32)
    h = jnp.maximum(h + b2_ref[...], 0.0)
    out_ref[...] = jnp.dot(w3_ref[...], h, preferred_element_type=jnp.float32) + b3_ref[...]


def _tc_mlp_t(encT, W1t, b1, W2t, b2, W3t, b3):
    BB = 2048
    grid = (BH // BB,)
    return pl.pallas_call(
        _mlp_body,
        grid=grid,
        in_specs=[
            pl.BlockSpec((IN_DIM, BB), lambda i: (0, i)),
            pl.BlockSpec((H1, IN_DIM), lambda i: (0, 0)),
            pl.BlockSpec((H1, 1), lambda i: (0, 0)),
            pl.BlockSpec((H2, H1), lambda i: (0, 0)),
            pl.BlockSpec((H2, 1), lambda i: (0, 0)),
            pl.BlockSpec((1, H2), lambda i: (0, 0)),
            pl.BlockSpec((1, 1), lambda i: (0, 0)),
        ],
        out_specs=pl.BlockSpec((1, BB), lambda i: (0, i)),
        out_shape=jax.ShapeDtypeStruct((1, BH), jnp.float32),
    )(encT, W1t, b1, W2t, b2, W3t, b3)


def kernel(x, emb_tables, W1, b1, W2, b2, W3, b3):
    tableT = emb_tables.transpose(0, 2, 1)
    xt = x.T
    W1t, b1c = W1.T, b1.reshape(H1, 1)
    W2t, b2c = W2.T, b2.reshape(H2, 1)
    W3t, b3c = W3.T, b3.reshape(1, 1)
    outs = []
    for k in range(2):
        xt3 = xt[:, k * BH:(k + 1) * BH].reshape(NV, BH // GROUP, GROUP)
        encT = _sc_gather(tableT, xt3).reshape(IN_DIM, BH)
        outs.append(_tc_mlp_t(encT, W1t, b1c, W2t, b2c, W3t, b3c))
    return jnp.concatenate(outs, axis=1).reshape(B, 1)


# speedup vs baseline: 1.0244x; 1.0244x over previous
import functools

import jax
import jax.numpy as jnp
from jax import lax
from jax.experimental import pallas as pl
from jax.experimental.pallas import tpu as pltpu
from jax.experimental.pallas import tpu_sc as plsc

B = 16384
NV = 26
CARD = 100000
ED = 16
IN_DIM = NV * ED
H1 = 128
H2 = 64

NC = 2
NS = 16
NW = NC * NS
BW = B // NW
GROUP = 512
CPF = BW // GROUP
NG = NV * CPF


def _sc_gather_body(table_hbm, xt_hbm, out_hbm, idx_v, rows_v, sem0, sem1):
    wid = lax.axis_index("s") * NC + lax.axis_index("c")
    b0 = wid * BW

    pltpu.sync_copy(xt_hbm.at[:, pl.ds(wid * CPF, CPF)], idx_v)

    def fire(v, c, buf_ref, sem):
        for e in range(ED):
            pltpu.make_async_copy(
                table_hbm.at[v, e].at[idx_v.at[v, c]], buf_ref.at[e], sem
            ).start()

    def drain(v, c, buf_ref, sem):
        for e in range(ED):
            pltpu.make_async_copy(
                table_hbm.at[v, e].at[idx_v.at[v, c]], buf_ref.at[e], sem
            ).wait()

    def write(v, c, buf_ref):
        pltpu.sync_copy(
            buf_ref, out_hbm.at[v, :, pl.ds(b0 + c * GROUP, GROUP)]
        )

    fire(0, 0, rows_v.at[0], sem0)

    def pair(p, _):
        g0 = p * 2
        v0, c0 = lax.div(g0, CPF), lax.rem(g0, CPF)
        v1, c1 = lax.div(g0 + 1, CPF), lax.rem(g0 + 1, CPF)
        drain(v0, c0, rows_v.at[0], sem0)
        fire(v1, c1, rows_v.at[1], sem1)
        write(v0, c0, rows_v.at[0])
        drain(v1, c1, rows_v.at[1], sem1)

        @pl.when(p < NG // 2 - 1)
        def _():
            v2, c2 = lax.div(g0 + 2, CPF), lax.rem(g0 + 2, CPF)
            fire(v2, c2, rows_v.at[0], sem0)

        write(v1, c1, rows_v.at[1])
        return 0

    lax.fori_loop(0, NG // 2, pair, 0)


def _sc_gather(tableT, xt3):
    mesh = plsc.VectorSubcoreMesh(
        core_axis_name="c", subcore_axis_name="s", num_cores=NC, num_subcores=NS
    )
    fn = pl.kernel(
        _sc_gather_body,
        out_type=jax.ShapeDtypeStruct((NV, ED, B), jnp.float32),
        mesh=mesh,
        compiler_params=pltpu.CompilerParams(use_tc_tiling_on_sc=False),
        scratch_types=[
            pltpu.VMEM((NV, CPF, GROUP), jnp.int32),
            pltpu.VMEM((2, ED, GROUP), jnp.float32),
            pltpu.SemaphoreType.DMA,
            pltpu.SemaphoreType.DMA,
        ],
    )
    return fn(tableT, xt3)


def _mlp_body(enc_ref, w1_ref, b1_ref, w2_ref, b2_ref, w3_ref, b3_ref, out_ref):
    h = jnp.dot(w1_ref[...], enc_ref[...], preferred_element_type=jnp.float32)
    h = jnp.maximum(h + b1_ref[...], 0.0)
    h = jnp.dot(w2_ref[...], h, preferred_element_type=jnp.float32)
    h = jnp.maximum(h + b2_ref[...], 0.0)
    out_ref[...] = jnp.dot(w3_ref[...], h, preferred_element_type=jnp.float32) + b3_ref[...]


def _tc_mlp_t(encT, W1t, b1, W2t, b2, W3t, b3):
    BB = 2048
    grid = (B // BB,)
    return pl.pallas_call(
        _mlp_body,
        grid=grid,
        in_specs=[
            pl.BlockSpec((IN_DIM, BB), lambda i: (0, i)),
            pl.BlockSpec((H1, IN_DIM), lambda i: (0, 0)),
            pl.BlockSpec((H1, 1), lambda i: (0, 0)),
            pl.BlockSpec((H2, H1), lambda i: (0, 0)),
            pl.BlockSpec((H2, 1), lambda i: (0, 0)),
            pl.BlockSpec((1, H2), lambda i: (0, 0)),
            pl.BlockSpec((1, 1), lambda i: (0, 0)),
        ],
        out_specs=pl.BlockSpec((1, BB), lambda i: (0, i)),
        out_shape=jax.ShapeDtypeStruct((1, B), jnp.float32),
    )(encT, W1t, b1, W2t, b2, W3t, b3)


def kernel(x, emb_tables, W1, b1, W2, b2, W3, b3):
    tableT = emb_tables.transpose(0, 2, 1)
    xt3 = x.T.reshape(NV, B // GROUP, GROUP)
    encT = _sc_gather(tableT, xt3).reshape(IN_DIM, B)
    outT = _tc_mlp_t(
        encT,
        W1.T, b1.reshape(H1, 1),
        W2.T, b2.reshape(H2, 1),
        W3.T, b3.reshape(1, 1),
    )
    return outT.reshape(B, 1)
